# baseline (speedup 1.0000x reference)
import functools

import jax
import jax.numpy as jnp
from jax import lax
from jax.experimental import pallas as pl
from jax.experimental.pallas import tpu as pltpu
from jax.experimental.pallas import tpu_sc as plsc

N = 10000
D = 256
E = 160000

NC = 2
NS = 16
NW = NC * NS
B = 128
NB = 40
EP = NW * NB * B
ER = EP // B

NP = 10240
TB = NP // NW
PEND = 6144
B2 = 48
CH = 32
NCH = ER // CH
PD = NP
HH = NP + 256

R = 256
G = NP // R
RL = 200
GL = N // RL

_mesh = plsc.VectorSubcoreMesh(core_axis_name="c", subcore_axis_name="s")
_sc_params = pltpu.CompilerParams(needs_layout_passes=False)


def _scan_body(s_hbm, d_hbm, pend_out, cnt_out, deg_out,
               sidx_c, didx_c, pend_s, pend_d, hist_v, cnt_v):
    c = lax.axis_index("c")
    s = lax.axis_index("s")
    wid = s * NC + c
    base = wid * TB
    ones16 = jnp.ones((16,), jnp.float32)
    zero16f = jnp.zeros((16,), jnp.float32)
    pads16 = jnp.full((16,), N, jnp.int32)
    zero16i = jnp.zeros((16,), jnp.int32)

    def zh(i, carry):
        hist_v[pl.ds(i * 16, 16)] = zero16f
        return carry

    lax.fori_loop(0, TB // 16, zh, 0)

    def pi(i, carry):
        pend_s[pl.ds(i * 16, 16)] = pads16
        pend_d[pl.ds(i * 16, 16)] = zero16i
        return carry

    lax.fori_loop(0, PEND // 16, pi, 0)

    def scan_chunk(ch, n):
        pltpu.sync_copy(s_hbm.at[pl.ds(ch * CH, CH)], sidx_c)
        pltpu.sync_copy(d_hbm.at[pl.ds(ch * CH, CH)], didx_c)

        def vecrow(j, n):
            for k in range(B // 16):
                dd = didx_c[j, pl.ds(k * 16, 16)]
                ss = sidx_c[j, pl.ds(k * 16, 16)]
                loc = dd - base
                ok = (loc >= 0) & (loc < TB)
                plsc.store_compressed(pend_s.at[pl.ds(n, 16)], ss, mask=ok)
                plsc.store_compressed(pend_d.at[pl.ds(n, 16)], loc * 16,
                                      mask=ok)
                plsc.addupdate_scatter(hist_v, [jnp.where(ok, loc, 0)], ones16,
                                       mask=ok)
                n = n + plsc.all_reduce_population_count(ok)[0]
            return n

        return lax.fori_loop(0, CH, vecrow, n)

    n = lax.fori_loop(0, NCH, scan_chunk, jnp.int32(0))
    cnt_v[pl.ds(0, 16)] = jnp.broadcast_to(n, (16,))
    pltpu.sync_copy(pend_s, pend_out.at[pl.ds(wid * PEND, PEND)])
    pltpu.sync_copy(pend_d, pend_out.at[pl.ds(NW * PEND + wid * PEND, PEND)])
    pltpu.sync_copy(cnt_v, cnt_out.at[pl.ds(wid * 16, 16)])
    pltpu.sync_copy(hist_v, deg_out.at[pl.ds(base, TB)])


_scan_kernel = pl.kernel(
    _scan_body,
    out_type=(
        jax.ShapeDtypeStruct((2 * NW * PEND,), jnp.int32),
        jax.ShapeDtypeStruct((NW * 16,), jnp.int32),
        jax.ShapeDtypeStruct((NP,), jnp.float32),
    ),
    mesh=_mesh,
    compiler_params=_sc_params,
    scratch_types=[
        pltpu.VMEM((CH, B), jnp.int32),
        pltpu.VMEM((CH, B), jnp.int32),
        pltpu.VMEM((PEND,), jnp.int32),
        pltpu.VMEM((PEND,), jnp.int32),
        pltpu.VMEM((TB,), jnp.float32),
        pltpu.VMEM((16,), jnp.int32),
    ],
)


def _acc_body(u_hbm, pend_hbm, cnt_hbm, out_hbm,
              pend_s, pend_d, rows_a, rows_b, cnt_v, accs, stage, sema, semb):
    c = lax.axis_index("c")
    s = lax.axis_index("s")
    wid = s * NC + c
    base = wid * TB
    zero16f = jnp.zeros((16,), jnp.float32)

    def za(i, carry):
        for a in accs:
            a[pl.ds(i * 16, 16)] = zero16f
        return carry

    lax.fori_loop(0, TB, za, 0)
    pltpu.sync_copy(pend_hbm.at[pl.ds(wid * PEND, PEND)], pend_s)
    pltpu.sync_copy(pend_hbm.at[pl.ds(NW * PEND + wid * PEND, PEND)], pend_d)
    pltpu.sync_copy(cnt_hbm.at[pl.ds(wid * 16, 16)], cnt_v)
    n = cnt_v[pl.ds(0, 16)][0]
    nb2 = (n + (2 * B2 - 1)) // (2 * B2)
    bmax = PEND // B2 - 1

    def start(b, buf, sem):
        bs = jnp.minimum(b, bmax) * B2
        return pltpu.async_copy(u_hbm.at[pend_s.at[pl.ds(bs, B2)]], buf, sem)

    def wait(buf, sem):
        pltpu.make_async_copy(u_hbm.at[pend_s.at[pl.ds(0, B2)]], buf, sem).wait()

    def acc(b, buf):
        def grp(g, carry2):
            rv = pend_d[pl.ds(b * B2 + g * 16, 16)]
            for i in range(16):
                ro = rv[i]
                e = g * 16 + i
                for k in range(D // 16):
                    accs[k][pl.ds(ro, 16)] = (
                        accs[k][pl.ds(ro, 16)] + buf[e, pl.ds(k * 16, 16)])
            return carry2

        lax.fori_loop(0, B2 // 16, grp, 0)

    start(0, rows_a, sema)

    def pair(bp, carry):
        b0 = 2 * bp
        wait(rows_a, sema)
        start(b0 + 1, rows_b, semb)
        acc(b0, rows_a)
        wait(rows_b, semb)
        start(b0 + 2, rows_a, sema)
        acc(b0 + 1, rows_b)
        return carry

    lax.fori_loop(0, nb2, pair, 0)
    wait(rows_a, sema)

    for cidx in range(TB // 32):
        def row(r, carry2, cidx=cidx):
            for k in range(D // 16):
                stage[r, pl.ds(k * 16, 16)] = accs[k][pl.ds((cidx * 32 + r) * 16, 16)]
            return carry2

        lax.fori_loop(0, 32, row, 0)
        pltpu.sync_copy(stage, out_hbm.at[pl.ds(base + cidx * 32, 32)])


_acc_kernel = pl.kernel(
    _acc_body,
    out_type=jax.ShapeDtypeStruct((NP, D), jnp.float32),
    mesh=_mesh,
    compiler_params=_sc_params,
    scratch_types=[
        pltpu.VMEM((PEND,), jnp.int32),
        pltpu.VMEM((PEND,), jnp.int32),
        pltpu.VMEM((B2, D), jnp.float32),
        pltpu.VMEM((B2, D), jnp.float32),
        pltpu.VMEM((16,), jnp.int32),
        [pltpu.VMEM((TB * 16,), jnp.float32) for _ in range(D // 16)],
        pltpu.VMEM((32, D), jnp.float32),
        pltpu.SemaphoreType.DMA,
        pltpu.SemaphoreType.DMA,
    ],
)


def _dinv_body(h_ref, o_ref):
    deg = h_ref[...] + 1.0
    dv = lax.rsqrt(deg)
    eye = jnp.eye(2, dtype=jnp.float32)
    o_ref[...] = lax.dot_general(dv, eye, (((0,), (0,)), ((), ())),
                                 preferred_element_type=jnp.float32)


def _dinv_call(degs):
    return pl.pallas_call(
        _dinv_body,
        grid=(1,),
        in_specs=[pl.BlockSpec((2, NP), lambda i: (0, 0))],
        out_specs=pl.BlockSpec((NP, 2), lambda i: (0, 0)),
        out_shape=jax.ShapeDtypeStruct((NP, 2), jnp.float32),
    )(degs)


def _mm1_body(view, x_ref, w_ref, dv_ref, o_ref):
    dv = dv_ref[...][:, view:view + 1]
    h = jnp.dot(x_ref[...], w_ref[...], preferred_element_type=jnp.float32)
    o_ref[...] = h * dv


def _mm1_call(x, w, dinv, view):
    return pl.pallas_call(
        functools.partial(_mm1_body, view),
        grid=(G,),
        in_specs=[
            pl.BlockSpec((R, D), lambda i: (i, 0)),
            pl.BlockSpec((D, D), lambda i: (0, 0)),
            pl.BlockSpec((R, 2), lambda i: (i, 0)),
        ],
        out_specs=pl.BlockSpec((R, D), lambda i: (i, 0)),
        out_shape=jax.ShapeDtypeStruct((NP, D), jnp.float32),
    )(x, w, dinv)


def _mm2_body(view, acc_ref, u_ref, w_ref, b_ref, dv_ref, o_ref):
    i = pl.program_id(0)
    dv = dv_ref[...][:, view:view + 1]
    z = dv * (acc_ref[...] + u_ref[...]) + b_ref[...]
    h = jnp.dot(z, w_ref[...], preferred_element_type=jnp.float32) * dv
    row = lax.broadcasted_iota(jnp.int32, (R, 1), 0) + i * R
    o_ref[...] = jnp.where(row < N, h, 0.0)


def _mm2_call(acc, u, w, b, dinv, view):
    return pl.pallas_call(
        functools.partial(_mm2_body, view),
        grid=(G,),
        in_specs=[
            pl.BlockSpec((R, D), lambda i: (i, 0)),
            pl.BlockSpec((R, D), lambda i: (i, 0)),
            pl.BlockSpec((D, D), lambda i: (0, 0)),
            pl.BlockSpec((1, D), lambda i: (0, 0)),
            pl.BlockSpec((R, 2), lambda i: (i, 0)),
        ],
        out_specs=pl.BlockSpec((R, D), lambda i: (i, 0)),
        out_shape=jax.ShapeDtypeStruct((NP, D), jnp.float32),
    )(acc, u, w, b, dinv)


def _fin_body(view, acc_ref, u_ref, b_ref, g_ref, bn_ref, dv_ref, o_ref):
    dv = dv_ref[...][:, view:view + 1]
    z = dv * (acc_ref[...] + u_ref[...]) + b_ref[...]
    mu = jnp.mean(z, axis=1, keepdims=True)
    zc = z - mu
    var = jnp.mean(zc * zc, axis=1, keepdims=True)
    o_ref[...] = zc * lax.rsqrt(var + 1e-5) * g_ref[...] + bn_ref[...]


def _fin_call(acc, u, b, g, bn, dinv, view):
    return pl.pallas_call(
        functools.partial(_fin_body, view),
        grid=(G,),
        in_specs=[
            pl.BlockSpec((R, D), lambda i: (i, 0)),
            pl.BlockSpec((R, D), lambda i: (i, 0)),
            pl.BlockSpec((1, D), lambda i: (0, 0)),
            pl.BlockSpec((1, D), lambda i: (0, 0)),
            pl.BlockSpec((1, D), lambda i: (0, 0)),
            pl.BlockSpec((R, 2), lambda i: (i, 0)),
        ],
        out_specs=pl.BlockSpec((R, D), lambda i: (i, 0)),
        out_shape=jax.ShapeDtypeStruct((NP, D), jnp.float32),
    )(acc, u, b, g, bn, dinv)


def _loss_body(v1_ref, v2_ref, wp_ref, bp_ref, gp_ref, b2_ref, o_ref):
    i = pl.program_id(0)
    v1 = v1_ref[...]
    v2 = v2_ref[...]

    def pred(v):
        h = jnp.dot(v, wp_ref[...], preferred_element_type=jnp.float32)
        h = h + bp_ref[...]
        mu = jnp.mean(h, axis=1, keepdims=True)
        hc = h - mu
        var = jnp.mean(hc * hc, axis=1, keepdims=True)
        return jnp.maximum(hc * lax.rsqrt(var + 1e-5) * gp_ref[...] + b2_ref[...], 0.0)

    def nrm(x):
        n = jnp.sqrt(jnp.sum(x * x, axis=1, keepdims=True))
        return x / jnp.maximum(n, 1e-12)

    p1 = pred(v1)
    p2 = pred(v2)
    l1 = 2.0 - 2.0 * jnp.sum(nrm(p1) * nrm(v2), axis=1)
    l2 = 2.0 - 2.0 * jnp.sum(nrm(p2) * nrm(v1), axis=1)
    part = jnp.reshape(jnp.sum(l1 + l2) * (1.0 / N), (1, 1))

    @pl.when(i == 0)
    def _():
        o_ref[...] = jnp.zeros_like(o_ref)

    o_ref[...] += part


def _loss_call(v1, v2, wp, bp, gp, b2):
    return pl.pallas_call(
        _loss_body,
        grid=(GL,),
        in_specs=[
            pl.BlockSpec((RL, D), lambda i: (i, 0)),
            pl.BlockSpec((RL, D), lambda i: (i, 0)),
            pl.BlockSpec((D, D), lambda i: (0, 0)),
            pl.BlockSpec((1, D), lambda i: (0, 0)),
            pl.BlockSpec((1, D), lambda i: (0, 0)),
            pl.BlockSpec((1, D), lambda i: (0, 0)),
        ],
        out_specs=pl.BlockSpec((1, 1), lambda i: (0, 0)),
        out_shape=jax.ShapeDtypeStruct((1, 1), jnp.float32),
    )(v1, v2, wp, bp, gp, b2)


def _prep_edges(ei):
    pad = EP - E
    s = jnp.concatenate([ei[0], jnp.full((pad,), N, jnp.int32)]).reshape(ER, B)
    d = jnp.concatenate([ei[1], jnp.full((pad,), PD, jnp.int32)]).reshape(ER, B)
    return s, d


def _encode(x, pend, cnt, dinv, view, W1, b1, W2, b2, gn, bn):
    u1 = _mm1_call(x, W1, dinv, view)
    acc1 = _acc_kernel(u1, pend, cnt)
    u2 = _mm2_call(acc1, u1, W2, b1, dinv, view)
    acc2 = _acc_kernel(u2, pend, cnt)
    return _fin_call(acc2, u2, b2, gn, bn, dinv, view)


def kernel(x1, x2, edge_index_v1, edge_index_v2, W1, b1, W2, b2, gn, bn,
           Wp, bp, gp, bp2):
    s1, d1 = _prep_edges(edge_index_v1)
    s2, d2 = _prep_edges(edge_index_v2)
    xpad = jnp.zeros((NP - N, D), jnp.float32)
    x1p = jnp.concatenate([x1, xpad])
    x2p = jnp.concatenate([x2, xpad])
    b1r = b1.reshape(1, D)
    b2r = b2.reshape(1, D)
    gnr = gn.reshape(1, D)
    bnr = bn.reshape(1, D)
    bpr = bp.reshape(1, D)
    gpr = gp.reshape(1, D)
    bp2r = bp2.reshape(1, D)

    pend1, cnt1, deg1 = _scan_kernel(s1, d1)
    pend2, cnt2, deg2 = _scan_kernel(s2, d2)
    dinv = _dinv_call(jnp.stack([deg1, deg2]))

    v1p = _encode(x1p, pend1, cnt1, dinv, 0, W1, b1r, W2, b2r, gnr, bnr)
    v2p = _encode(x2p, pend2, cnt2, dinv, 1, W1, b1r, W2, b2r, gnr, bnr)
    v1_rep = v1p[:N]
    v2_rep = v2p[:N]

    loss = _loss_call(v1_rep, v2_rep, Wp, bpr, gpr, bp2r)
    return (v1_rep, v2_rep, loss[0, 0])


# speedup vs baseline: 1.0460x; 1.0460x over previous
import functools

import jax
import jax.numpy as jnp
from jax import lax
from jax.experimental import pallas as pl
from jax.experimental.pallas import tpu as pltpu
from jax.experimental.pallas import tpu_sc as plsc

N = 10000
D = 256
E = 160000

NC = 2
NS = 16
NW = NC * NS
B = 128
NB = 40
EP = NW * NB * B
ER = EP // B

NP = 10240
TB = NP // NW
PEND = 6144
B2 = 32
CH = 32
NCH = ER // CH
PD = NP
HH = NP + 256

R = 256
G = NP // R
RL = 200
GL = N // RL

_mesh = plsc.VectorSubcoreMesh(core_axis_name="c", subcore_axis_name="s")
_sc_params = pltpu.CompilerParams(needs_layout_passes=False)


def _scan_body(s_hbm, d_hbm, pend_out, cnt_out, deg_out,
               sidx_c, didx_c, pend_s, pend_d, hist_v, cnt_v):
    c = lax.axis_index("c")
    s = lax.axis_index("s")
    wid = s * NC + c
    base = wid * TB
    ones16 = jnp.ones((16,), jnp.float32)
    zero16f = jnp.zeros((16,), jnp.float32)
    pads16 = jnp.full((16,), N, jnp.int32)
    zero16i = jnp.zeros((16,), jnp.int32)

    def zh(i, carry):
        hist_v[pl.ds(i * 16, 16)] = zero16f
        return carry

    lax.fori_loop(0, TB // 16, zh, 0)

    def pi(i, carry):
        pend_s[pl.ds(i * 16, 16)] = pads16
        pend_d[pl.ds(i * 16, 16)] = zero16i
        return carry

    lax.fori_loop(0, PEND // 16, pi, 0)

    def scan_chunk(ch, n):
        pltpu.sync_copy(s_hbm.at[pl.ds(ch * CH, CH)], sidx_c)
        pltpu.sync_copy(d_hbm.at[pl.ds(ch * CH, CH)], didx_c)

        def vecrow(j, n):
            for k in range(B // 16):
                dd = didx_c[j, pl.ds(k * 16, 16)]
                ss = sidx_c[j, pl.ds(k * 16, 16)]
                loc = dd - base
                ok = (loc >= 0) & (loc < TB)
                plsc.store_compressed(pend_s.at[pl.ds(n, 16)], ss, mask=ok)
                plsc.store_compressed(pend_d.at[pl.ds(n, 16)], loc * 16,
                                      mask=ok)
                plsc.addupdate_scatter(hist_v, [jnp.where(ok, loc, 0)], ones16,
                                       mask=ok)
                n = n + plsc.all_reduce_population_count(ok)[0]
            return n

        return lax.fori_loop(0, CH, vecrow, n)

    n = lax.fori_loop(0, NCH, scan_chunk, jnp.int32(0))
    cnt_v[pl.ds(0, 16)] = jnp.broadcast_to(n, (16,))
    pltpu.sync_copy(pend_s, pend_out.at[pl.ds(wid * PEND, PEND)])
    pltpu.sync_copy(pend_d, pend_out.at[pl.ds(NW * PEND + wid * PEND, PEND)])
    pltpu.sync_copy(cnt_v, cnt_out.at[pl.ds(wid * 16, 16)])
    pltpu.sync_copy(hist_v, deg_out.at[pl.ds(base, TB)])


_scan_kernel = pl.kernel(
    _scan_body,
    out_type=(
        jax.ShapeDtypeStruct((2 * NW * PEND,), jnp.int32),
        jax.ShapeDtypeStruct((NW * 16,), jnp.int32),
        jax.ShapeDtypeStruct((NP,), jnp.float32),
    ),
    mesh=_mesh,
    compiler_params=_sc_params,
    scratch_types=[
        pltpu.VMEM((CH, B), jnp.int32),
        pltpu.VMEM((CH, B), jnp.int32),
        pltpu.VMEM((PEND,), jnp.int32),
        pltpu.VMEM((PEND,), jnp.int32),
        pltpu.VMEM((TB,), jnp.float32),
        pltpu.VMEM((16,), jnp.int32),
    ],
)


def _acc_body(u_hbm, pend_hbm, cnt_hbm, out_hbm,
              pend_s, pend_d, rows_a, rows_b, cnt_v, accs, stage, sema, semb):
    c = lax.axis_index("c")
    s = lax.axis_index("s")
    wid = s * NC + c
    base = wid * TB
    zero16f = jnp.zeros((16,), jnp.float32)

    def za(i, carry):
        for a in accs:
            a[pl.ds(i * 16, 16)] = zero16f
        return carry

    lax.fori_loop(0, TB, za, 0)
    pltpu.sync_copy(pend_hbm.at[pl.ds(wid * PEND, PEND)], pend_s)
    pltpu.sync_copy(pend_hbm.at[pl.ds(NW * PEND + wid * PEND, PEND)], pend_d)
    pltpu.sync_copy(cnt_hbm.at[pl.ds(wid * 16, 16)], cnt_v)
    n = cnt_v[pl.ds(0, 16)][0]
    nb2 = (n + (2 * B2 - 1)) // (2 * B2)
    bmax = PEND // B2 - 1

    def start(b, buf, sem):
        bs = jnp.minimum(b, bmax) * B2
        return pltpu.async_copy(u_hbm.at[pend_s.at[pl.ds(bs, B2)]], buf, sem)

    def wait(buf, sem):
        pltpu.make_async_copy(u_hbm.at[pend_s.at[pl.ds(0, B2)]], buf, sem).wait()

    def acc(b, buf):
        def grp(g, carry2):
            rv = pend_d[pl.ds(b * B2 + g * 16, 16)]
            for i in range(16):
                ro = rv[i]
                e = g * 16 + i
                for k in range(D // 16):
                    accs[k][pl.ds(ro, 16)] = (
                        accs[k][pl.ds(ro, 16)] + buf[e, pl.ds(k * 16, 16)])
            return carry2

        lax.fori_loop(0, B2 // 16, grp, 0)

    start(0, rows_a, sema)

    def pair(bp, carry):
        b0 = 2 * bp
        wait(rows_a, sema)
        start(b0 + 1, rows_b, semb)
        acc(b0, rows_a)
        wait(rows_b, semb)
        start(b0 + 2, rows_a, sema)
        acc(b0 + 1, rows_b)
        return carry

    lax.fori_loop(0, nb2, pair, 0)
    wait(rows_a, sema)

    for cidx in range(TB // 40):
        def row(r, carry2, cidx=cidx):
            for k in range(D // 16):
                stage[r, pl.ds(k * 16, 16)] = accs[k][pl.ds((cidx * 40 + r) * 16, 16)]
            return carry2

        lax.fori_loop(0, 40, row, 0)
        pltpu.sync_copy(stage, out_hbm.at[pl.ds(base + cidx * 40, 40)])


_acc_kernel = pl.kernel(
    _acc_body,
    out_type=jax.ShapeDtypeStruct((NP, D), jnp.float32),
    mesh=_mesh,
    compiler_params=_sc_params,
    scratch_types=[
        pltpu.VMEM((PEND,), jnp.int32),
        pltpu.VMEM((PEND,), jnp.int32),
        pltpu.VMEM((B2, D), jnp.float32),
        pltpu.VMEM((B2, D), jnp.float32),
        pltpu.VMEM((16,), jnp.int32),
        [pltpu.VMEM((TB * 16,), jnp.float32) for _ in range(D // 16)],
        pltpu.VMEM((40, D), jnp.float32),
        pltpu.SemaphoreType.DMA,
        pltpu.SemaphoreType.DMA,
    ],
)


def _dinv_body(h_ref, o_ref):
    deg = h_ref[...] + 1.0
    dv = lax.rsqrt(deg)
    eye = jnp.eye(2, dtype=jnp.float32)
    o_ref[...] = lax.dot_general(dv, eye, (((0,), (0,)), ((), ())),
                                 preferred_element_type=jnp.float32)


def _dinv_call(degs):
    return pl.pallas_call(
        _dinv_body,
        grid=(1,),
        in_specs=[pl.BlockSpec((2, NP), lambda i: (0, 0))],
        out_specs=pl.BlockSpec((NP, 2), lambda i: (0, 0)),
        out_shape=jax.ShapeDtypeStruct((NP, 2), jnp.float32),
    )(degs)


def _mm1_body(view, x_ref, w_ref, dv_ref, o_ref):
    dv = dv_ref[...][:, view:view + 1]
    h = jnp.dot(x_ref[...], w_ref[...], preferred_element_type=jnp.float32)
    o_ref[...] = h * dv


def _mm1_call(x, w, dinv, view):
    return pl.pallas_call(
        functools.partial(_mm1_body, view),
        grid=(G,),
        in_specs=[
            pl.BlockSpec((R, D), lambda i: (i, 0)),
            pl.BlockSpec((D, D), lambda i: (0, 0)),
            pl.BlockSpec((R, 2), lambda i: (i, 0)),
        ],
        out_specs=pl.BlockSpec((R, D), lambda i: (i, 0)),
        out_shape=jax.ShapeDtypeStruct((NP, D), jnp.float32),
    )(x, w, dinv)


def _mm2_body(view, acc_ref, u_ref, w_ref, b_ref, dv_ref, o_ref):
    i = pl.program_id(0)
    dv = dv_ref[...][:, view:view + 1]
    z = dv * (acc_ref[...] + u_ref[...]) + b_ref[...]
    h = jnp.dot(z, w_ref[...], preferred_element_type=jnp.float32) * dv
    row = lax.broadcasted_iota(jnp.int32, (R, 1), 0) + i * R
    o_ref[...] = jnp.where(row < N, h, 0.0)


def _mm2_call(acc, u, w, b, dinv, view):
    return pl.pallas_call(
        functools.partial(_mm2_body, view),
        grid=(G,),
        in_specs=[
            pl.BlockSpec((R, D), lambda i: (i, 0)),
            pl.BlockSpec((R, D), lambda i: (i, 0)),
            pl.BlockSpec((D, D), lambda i: (0, 0)),
            pl.BlockSpec((1, D), lambda i: (0, 0)),
            pl.BlockSpec((R, 2), lambda i: (i, 0)),
        ],
        out_specs=pl.BlockSpec((R, D), lambda i: (i, 0)),
        out_shape=jax.ShapeDtypeStruct((NP, D), jnp.float32),
    )(acc, u, w, b, dinv)


def _fin_body(view, acc_ref, u_ref, b_ref, g_ref, bn_ref, dv_ref, o_ref):
    dv = dv_ref[...][:, view:view + 1]
    z = dv * (acc_ref[...] + u_ref[...]) + b_ref[...]
    mu = jnp.mean(z, axis=1, keepdims=True)
    zc = z - mu
    var = jnp.mean(zc * zc, axis=1, keepdims=True)
    o_ref[...] = zc * lax.rsqrt(var + 1e-5) * g_ref[...] + bn_ref[...]


def _fin_call(acc, u, b, g, bn, dinv, view):
    return pl.pallas_call(
        functools.partial(_fin_body, view),
        grid=(G,),
        in_specs=[
            pl.BlockSpec((R, D), lambda i: (i, 0)),
            pl.BlockSpec((R, D), lambda i: (i, 0)),
            pl.BlockSpec((1, D), lambda i: (0, 0)),
            pl.BlockSpec((1, D), lambda i: (0, 0)),
            pl.BlockSpec((1, D), lambda i: (0, 0)),
            pl.BlockSpec((R, 2), lambda i: (i, 0)),
        ],
        out_specs=pl.BlockSpec((R, D), lambda i: (i, 0)),
        out_shape=jax.ShapeDtypeStruct((NP, D), jnp.float32),
    )(acc, u, b, g, bn, dinv)


def _loss_body(v1_ref, v2_ref, wp_ref, bp_ref, gp_ref, b2_ref, o_ref):
    i = pl.program_id(0)
    v1 = v1_ref[...]
    v2 = v2_ref[...]

    def pred(v):
        h = jnp.dot(v, wp_ref[...], preferred_element_type=jnp.float32)
        h = h + bp_ref[...]
        mu = jnp.mean(h, axis=1, keepdims=True)
        hc = h - mu
        var = jnp.mean(hc * hc, axis=1, keepdims=True)
        return jnp.maximum(hc * lax.rsqrt(var + 1e-5) * gp_ref[...] + b2_ref[...], 0.0)

    def nrm(x):
        n = jnp.sqrt(jnp.sum(x * x, axis=1, keepdims=True))
        return x / jnp.maximum(n, 1e-12)

    p1 = pred(v1)
    p2 = pred(v2)
    l1 = 2.0 - 2.0 * jnp.sum(nrm(p1) * nrm(v2), axis=1)
    l2 = 2.0 - 2.0 * jnp.sum(nrm(p2) * nrm(v1), axis=1)
    part = jnp.reshape(jnp.sum(l1 + l2) * (1.0 / N), (1, 1))

    @pl.when(i == 0)
    def _():
        o_ref[...] = jnp.zeros_like(o_ref)

    o_ref[...] += part


def _loss_call(v1, v2, wp, bp, gp, b2):
    return pl.pallas_call(
        _loss_body,
        grid=(GL,),
        in_specs=[
            pl.BlockSpec((RL, D), lambda i: (i, 0)),
            pl.BlockSpec((RL, D), lambda i: (i, 0)),
            pl.BlockSpec((D, D), lambda i: (0, 0)),
            pl.BlockSpec((1, D), lambda i: (0, 0)),
            pl.BlockSpec((1, D), lambda i: (0, 0)),
            pl.BlockSpec((1, D), lambda i: (0, 0)),
        ],
        out_specs=pl.BlockSpec((1, 1), lambda i: (0, 0)),
        out_shape=jax.ShapeDtypeStruct((1, 1), jnp.float32),
    )(v1, v2, wp, bp, gp, b2)


def _prep_edges(ei):
    pad = EP - E
    s = jnp.concatenate([ei[0], jnp.full((pad,), N, jnp.int32)]).reshape(ER, B)
    d = jnp.concatenate([ei[1], jnp.full((pad,), PD, jnp.int32)]).reshape(ER, B)
    return s, d


def _encode(x, pend, cnt, dinv, view, W1, b1, W2, b2, gn, bn):
    u1 = _mm1_call(x, W1, dinv, view)
    acc1 = _acc_kernel(u1, pend, cnt)
    u2 = _mm2_call(acc1, u1, W2, b1, dinv, view)
    acc2 = _acc_kernel(u2, pend, cnt)
    return _fin_call(acc2, u2, b2, gn, bn, dinv, view)


def kernel(x1, x2, edge_index_v1, edge_index_v2, W1, b1, W2, b2, gn, bn,
           Wp, bp, gp, bp2):
    s1, d1 = _prep_edges(edge_index_v1)
    s2, d2 = _prep_edges(edge_index_v2)
    xpad = jnp.zeros((NP - N, D), jnp.float32)
    x1p = jnp.concatenate([x1, xpad])
    x2p = jnp.concatenate([x2, xpad])
    b1r = b1.reshape(1, D)
    b2r = b2.reshape(1, D)
    gnr = gn.reshape(1, D)
    bnr = bn.reshape(1, D)
    bpr = bp.reshape(1, D)
    gpr = gp.reshape(1, D)
    bp2r = bp2.reshape(1, D)

    pend1, cnt1, deg1 = _scan_kernel(s1, d1)
    pend2, cnt2, deg2 = _scan_kernel(s2, d2)
    dinv = _dinv_call(jnp.stack([deg1, deg2]))

    v1p = _encode(x1p, pend1, cnt1, dinv, 0, W1, b1r, W2, b2r, gnr, bnr)
    v2p = _encode(x2p, pend2, cnt2, dinv, 1, W1, b1r, W2, b2r, gnr, bnr)
    v1_rep = v1p[:N]
    v2_rep = v2p[:N]

    loss = _loss_call(v1_rep, v2_rep, Wp, bpr, gpr, bp2r)
    return (v1_rep, v2_rep, loss[0, 0])


# speedup vs baseline: 1.0641x; 1.0173x over previous
import functools

import jax
import jax.numpy as jnp
from jax import lax
from jax.experimental import pallas as pl
from jax.experimental.pallas import tpu as pltpu
from jax.experimental.pallas import tpu_sc as plsc

N = 10000
D = 256
E = 160000

NC = 2
NS = 16
NW = NC * NS
B = 128
NB = 40
EP = NW * NB * B
ER = EP // B

NP = 10240
TB = NP // NW
PEND = 6144
B2 = 32
CH = 64
NCH = ER // CH
PD = NP
HH = NP + 256

R = 256
G = NP // R
RL = 200
GL = N // RL

_mesh = plsc.VectorSubcoreMesh(core_axis_name="c", subcore_axis_name="s")
_sc_params = pltpu.CompilerParams(needs_layout_passes=False)


def _scan_body(s_hbm, d_hbm, pend_out, cnt_out, deg_out,
               sidx_c, didx_c, pend_s, pend_d, hist_v, cnt_v):
    c = lax.axis_index("c")
    s = lax.axis_index("s")
    wid = s * NC + c
    base = wid * TB
    ones16 = jnp.ones((16,), jnp.float32)
    zero16f = jnp.zeros((16,), jnp.float32)
    pads16 = jnp.full((16,), N, jnp.int32)
    zero16i = jnp.zeros((16,), jnp.int32)

    def zh(i, carry):
        hist_v[pl.ds(i * 16, 16)] = zero16f
        return carry

    lax.fori_loop(0, TB // 16, zh, 0)

    def pi(i, carry):
        pend_s[pl.ds(i * 16, 16)] = pads16
        pend_d[pl.ds(i * 16, 16)] = zero16i
        return carry

    lax.fori_loop(0, PEND // 16, pi, 0)

    def scan_chunk(ch, n):
        pltpu.sync_copy(s_hbm.at[pl.ds(ch * CH, CH)], sidx_c)
        pltpu.sync_copy(d_hbm.at[pl.ds(ch * CH, CH)], didx_c)

        def vecrow(j, n):
            for k in range(B // 16):
                dd = didx_c[j, pl.ds(k * 16, 16)]
                ss = sidx_c[j, pl.ds(k * 16, 16)]
                loc = dd - base
                ok = (loc >= 0) & (loc < TB)
                plsc.store_compressed(pend_s.at[pl.ds(n, 16)], ss, mask=ok)
                plsc.store_compressed(pend_d.at[pl.ds(n, 16)], loc * 16,
                                      mask=ok)
                plsc.addupdate_scatter(hist_v, [jnp.where(ok, loc, 0)], ones16,
                                       mask=ok)
                n = n + plsc.all_reduce_population_count(ok)[0]
            return n

        return lax.fori_loop(0, CH, vecrow, n)

    n = lax.fori_loop(0, NCH, scan_chunk, jnp.int32(0))
    cnt_v[pl.ds(0, 16)] = jnp.broadcast_to(n, (16,))
    pltpu.sync_copy(pend_s, pend_out.at[pl.ds(wid * PEND, PEND)])
    pltpu.sync_copy(pend_d, pend_out.at[pl.ds(NW * PEND + wid * PEND, PEND)])
    pltpu.sync_copy(cnt_v, cnt_out.at[pl.ds(wid * 16, 16)])
    pltpu.sync_copy(hist_v, deg_out.at[pl.ds(base, TB)])


_scan_kernel = pl.kernel(
    _scan_body,
    out_type=(
        jax.ShapeDtypeStruct((2 * NW * PEND,), jnp.int32),
        jax.ShapeDtypeStruct((NW * 16,), jnp.int32),
        jax.ShapeDtypeStruct((NP,), jnp.float32),
    ),
    mesh=_mesh,
    compiler_params=_sc_params,
    scratch_types=[
        pltpu.VMEM((CH, B), jnp.int32),
        pltpu.VMEM((CH, B), jnp.int32),
        pltpu.VMEM((PEND,), jnp.int32),
        pltpu.VMEM((PEND,), jnp.int32),
        pltpu.VMEM((TB,), jnp.float32),
        pltpu.VMEM((16,), jnp.int32),
    ],
)


def _acc_body(u_hbm, pend_hbm, cnt_hbm, out_hbm,
              pend_s, pend_d, rows_a, rows_b, cnt_v, accs, stage, sema, semb):
    c = lax.axis_index("c")
    s = lax.axis_index("s")
    wid = s * NC + c
    base = wid * TB
    zero16f = jnp.zeros((16,), jnp.float32)

    def za(i, carry):
        for a in accs:
            a[pl.ds(i * 16, 16)] = zero16f
        return carry

    lax.fori_loop(0, TB, za, 0)
    pltpu.sync_copy(pend_hbm.at[pl.ds(wid * PEND, PEND)], pend_s)
    pltpu.sync_copy(pend_hbm.at[pl.ds(NW * PEND + wid * PEND, PEND)], pend_d)
    pltpu.sync_copy(cnt_hbm.at[pl.ds(wid * 16, 16)], cnt_v)
    n = cnt_v[pl.ds(0, 16)][0]
    nb2 = (n + (2 * B2 - 1)) // (2 * B2)
    bmax = PEND // B2 - 1

    def start(b, buf, sem):
        bs = jnp.minimum(b, bmax) * B2
        return pltpu.async_copy(u_hbm.at[pend_s.at[pl.ds(bs, B2)]], buf, sem)

    def wait(buf, sem):
        pltpu.make_async_copy(u_hbm.at[pend_s.at[pl.ds(0, B2)]], buf, sem).wait()

    def acc(b, buf):
        def grp(g, carry2):
            rv = pend_d[pl.ds(b * B2 + g * 16, 16)]
            for i in range(16):
                ro = rv[i]
                e = g * 16 + i
                for k in range(D // 16):
                    accs[k][pl.ds(ro, 16)] = (
                        accs[k][pl.ds(ro, 16)] + buf[e, pl.ds(k * 16, 16)])
            return carry2

        lax.fori_loop(0, B2 // 16, grp, 0)

    start(0, rows_a, sema)

    def pair(bp, carry):
        b0 = 2 * bp
        wait(rows_a, sema)
        start(b0 + 1, rows_b, semb)
        acc(b0, rows_a)
        wait(rows_b, semb)
        start(b0 + 2, rows_a, sema)
        acc(b0 + 1, rows_b)
        return carry

    lax.fori_loop(0, nb2, pair, 0)
    wait(rows_a, sema)

    for cidx in range(TB // 40):
        def row(r, carry2, cidx=cidx):
            for k in range(D // 16):
                stage[r, pl.ds(k * 16, 16)] = accs[k][pl.ds((cidx * 40 + r) * 16, 16)]
            return carry2

        lax.fori_loop(0, 40, row, 0)
        pltpu.sync_copy(stage, out_hbm.at[pl.ds(base + cidx * 40, 40)])


_acc_kernel = pl.kernel(
    _acc_body,
    out_type=jax.ShapeDtypeStruct((NP, D), jnp.float32),
    mesh=_mesh,
    compiler_params=_sc_params,
    scratch_types=[
        pltpu.VMEM((PEND,), jnp.int32),
        pltpu.VMEM((PEND,), jnp.int32),
        pltpu.VMEM((B2, D), jnp.float32),
        pltpu.VMEM((B2, D), jnp.float32),
        pltpu.VMEM((16,), jnp.int32),
        [pltpu.VMEM((TB * 16,), jnp.float32) for _ in range(D // 16)],
        pltpu.VMEM((40, D), jnp.float32),
        pltpu.SemaphoreType.DMA,
        pltpu.SemaphoreType.DMA,
    ],
)


def _dinv_body(h_ref, o_ref):
    deg = h_ref[...] + 1.0
    dv = lax.rsqrt(deg)
    eye = jnp.eye(2, dtype=jnp.float32)
    o_ref[...] = lax.dot_general(dv, eye, (((0,), (0,)), ((), ())),
                                 preferred_element_type=jnp.float32)


def _dinv_call(degs):
    return pl.pallas_call(
        _dinv_body,
        grid=(1,),
        in_specs=[pl.BlockSpec((2, NP), lambda i: (0, 0))],
        out_specs=pl.BlockSpec((NP, 2), lambda i: (0, 0)),
        out_shape=jax.ShapeDtypeStruct((NP, 2), jnp.float32),
    )(degs)


def _mm1_body(view, x_ref, w_ref, dv_ref, o_ref):
    dv = dv_ref[...][:, view:view + 1]
    h = jnp.dot(x_ref[...], w_ref[...], preferred_element_type=jnp.float32)
    o_ref[...] = h * dv


def _mm1_call(x, w, dinv, view):
    return pl.pallas_call(
        functools.partial(_mm1_body, view),
        grid=(G,),
        in_specs=[
            pl.BlockSpec((R, D), lambda i: (i, 0)),
            pl.BlockSpec((D, D), lambda i: (0, 0)),
            pl.BlockSpec((R, 2), lambda i: (i, 0)),
        ],
        out_specs=pl.BlockSpec((R, D), lambda i: (i, 0)),
        out_shape=jax.ShapeDtypeStruct((NP, D), jnp.float32),
    )(x, w, dinv)


def _mm2_body(view, acc_ref, u_ref, w_ref, b_ref, dv_ref, o_ref):
    i = pl.program_id(0)
    dv = dv_ref[...][:, view:view + 1]
    z = dv * (acc_ref[...] + u_ref[...]) + b_ref[...]
    h = jnp.dot(z, w_ref[...], preferred_element_type=jnp.float32) * dv
    row = lax.broadcasted_iota(jnp.int32, (R, 1), 0) + i * R
    o_ref[...] = jnp.where(row < N, h, 0.0)


def _mm2_call(acc, u, w, b, dinv, view):
    return pl.pallas_call(
        functools.partial(_mm2_body, view),
        grid=(G,),
        in_specs=[
            pl.BlockSpec((R, D), lambda i: (i, 0)),
            pl.BlockSpec((R, D), lambda i: (i, 0)),
            pl.BlockSpec((D, D), lambda i: (0, 0)),
            pl.BlockSpec((1, D), lambda i: (0, 0)),
            pl.BlockSpec((R, 2), lambda i: (i, 0)),
        ],
        out_specs=pl.BlockSpec((R, D), lambda i: (i, 0)),
        out_shape=jax.ShapeDtypeStruct((NP, D), jnp.float32),
    )(acc, u, w, b, dinv)


def _fin_body(view, acc_ref, u_ref, b_ref, g_ref, bn_ref, dv_ref, o_ref):
    dv = dv_ref[...][:, view:view + 1]
    z = dv * (acc_ref[...] + u_ref[...]) + b_ref[...]
    mu = jnp.mean(z, axis=1, keepdims=True)
    zc = z - mu
    var = jnp.mean(zc * zc, axis=1, keepdims=True)
    o_ref[...] = zc * lax.rsqrt(var + 1e-5) * g_ref[...] + bn_ref[...]


def _fin_call(acc, u, b, g, bn, dinv, view):
    return pl.pallas_call(
        functools.partial(_fin_body, view),
        grid=(G,),
        in_specs=[
            pl.BlockSpec((R, D), lambda i: (i, 0)),
            pl.BlockSpec((R, D), lambda i: (i, 0)),
            pl.BlockSpec((1, D), lambda i: (0, 0)),
            pl.BlockSpec((1, D), lambda i: (0, 0)),
            pl.BlockSpec((1, D), lambda i: (0, 0)),
            pl.BlockSpec((R, 2), lambda i: (i, 0)),
        ],
        out_specs=pl.BlockSpec((R, D), lambda i: (i, 0)),
        out_shape=jax.ShapeDtypeStruct((NP, D), jnp.float32),
    )(acc, u, b, g, bn, dinv)


def _loss_body(v1_ref, v2_ref, wp_ref, bp_ref, gp_ref, b2_ref, o_ref):
    i = pl.program_id(0)
    v1 = v1_ref[...]
    v2 = v2_ref[...]

    def pred(v):
        h = jnp.dot(v, wp_ref[...], preferred_element_type=jnp.float32)
        h = h + bp_ref[...]
        mu = jnp.mean(h, axis=1, keepdims=True)
        hc = h - mu
        var = jnp.mean(hc * hc, axis=1, keepdims=True)
        return jnp.maximum(hc * lax.rsqrt(var + 1e-5) * gp_ref[...] + b2_ref[...], 0.0)

    def nrm(x):
        n = jnp.sqrt(jnp.sum(x * x, axis=1, keepdims=True))
        return x / jnp.maximum(n, 1e-12)

    p1 = pred(v1)
    p2 = pred(v2)
    l1 = 2.0 - 2.0 * jnp.sum(nrm(p1) * nrm(v2), axis=1)
    l2 = 2.0 - 2.0 * jnp.sum(nrm(p2) * nrm(v1), axis=1)
    part = jnp.reshape(jnp.sum(l1 + l2) * (1.0 / N), (1, 1))

    @pl.when(i == 0)
    def _():
        o_ref[...] = jnp.zeros_like(o_ref)

    o_ref[...] += part


def _loss_call(v1, v2, wp, bp, gp, b2):
    return pl.pallas_call(
        _loss_body,
        grid=(GL,),
        in_specs=[
            pl.BlockSpec((RL, D), lambda i: (i, 0)),
            pl.BlockSpec((RL, D), lambda i: (i, 0)),
            pl.BlockSpec((D, D), lambda i: (0, 0)),
            pl.BlockSpec((1, D), lambda i: (0, 0)),
            pl.BlockSpec((1, D), lambda i: (0, 0)),
            pl.BlockSpec((1, D), lambda i: (0, 0)),
        ],
        out_specs=pl.BlockSpec((1, 1), lambda i: (0, 0)),
        out_shape=jax.ShapeDtypeStruct((1, 1), jnp.float32),
    )(v1, v2, wp, bp, gp, b2)


def _prep_edges(ei):
    pad = EP - E
    s = jnp.concatenate([ei[0], jnp.full((pad,), N, jnp.int32)]).reshape(ER, B)
    d = jnp.concatenate([ei[1], jnp.full((pad,), PD, jnp.int32)]).reshape(ER, B)
    return s, d


def _encode(x, pend, cnt, dinv, view, W1, b1, W2, b2, gn, bn):
    u1 = _mm1_call(x, W1, dinv, view)
    acc1 = _acc_kernel(u1, pend, cnt)
    u2 = _mm2_call(acc1, u1, W2, b1, dinv, view)
    acc2 = _acc_kernel(u2, pend, cnt)
    return _fin_call(acc2, u2, b2, gn, bn, dinv, view)


def kernel(x1, x2, edge_index_v1, edge_index_v2, W1, b1, W2, b2, gn, bn,
           Wp, bp, gp, bp2):
    s1, d1 = _prep_edges(edge_index_v1)
    s2, d2 = _prep_edges(edge_index_v2)
    xpad = jnp.zeros((NP - N, D), jnp.float32)
    x1p = jnp.concatenate([x1, xpad])
    x2p = jnp.concatenate([x2, xpad])
    b1r = b1.reshape(1, D)
    b2r = b2.reshape(1, D)
    gnr = gn.reshape(1, D)
    bnr = bn.reshape(1, D)
    bpr = bp.reshape(1, D)
    gpr = gp.reshape(1, D)
    bp2r = bp2.reshape(1, D)

    pend1, cnt1, deg1 = _scan_kernel(s1, d1)
    pend2, cnt2, deg2 = _scan_kernel(s2, d2)
    dinv = _dinv_call(jnp.stack([deg1, deg2]))

    v1p = _encode(x1p, pend1, cnt1, dinv, 0, W1, b1r, W2, b2r, gnr, bnr)
    v2p = _encode(x2p, pend2, cnt2, dinv, 1, W1, b1r, W2, b2r, gnr, bnr)
    v1_rep = v1p[:N]
    v2_rep = v2p[:N]

    loss = _loss_call(v1_rep, v2_rep, Wp, bpr, gpr, bp2r)
    return (v1_rep, v2_rep, loss[0, 0])
